# Initial kernel scaffold; baseline (speedup 1.0000x reference)
import jax
import jax.numpy as jnp
from jax.experimental import pallas as pl


def kernel(x, emb_table):
    raise NotImplementedError("write your pallas kernel here")


# speedup vs baseline: 5.0800x; 5.0800x over previous
import functools

import jax
import jax.numpy as jnp
from jax import lax
from jax.experimental import pallas as pl
from jax.experimental.pallas import tpu as pltpu
from jax.experimental.pallas import tpu_sc as plsc


@functools.lru_cache(maxsize=None)
def _make_gather(n_total, n_var, d_model):
    info = plsc.get_sparse_core_info()
    nc, ns = info.num_cores, info.num_subcores
    nw = nc * ns

    chunk = 512
    per_w = n_total // nw
    n_chunks = per_w // chunk
    assert per_w * nw == n_total and n_chunks * chunk == per_w

    mesh = plsc.VectorSubcoreMesh(core_axis_name="c", subcore_axis_name="s")

    @functools.partial(
        pl.kernel,
        mesh=mesh,
        out_type=jax.ShapeDtypeStruct((n_total, d_model), jnp.float32),
        scratch_types=[
            pltpu.VMEM((chunk,), jnp.int32),
            pltpu.VMEM((chunk, d_model), jnp.float32),
            pltpu.VMEM_SHARED((n_var, d_model), jnp.float32),
            pltpu.SemaphoreType.DMA,
        ],
        compiler_params=pltpu.CompilerParams(use_tc_tiling_on_sc=False),
    )
    def gather_kernel(idx_hbm, table_hbm, out_hbm, idx_v, rows_v, table_s, sem):
        sid = lax.axis_index("s")
        wid = sid * nc + lax.axis_index("c")
        base = wid * per_w

        @pl.when(sid == 0)
        def _stage():
            pltpu.sync_copy(table_hbm, table_s)

        plsc.subcore_barrier()

        def body(i, carry):
            off = base + i * chunk
            pltpu.sync_copy(idx_hbm.at[pl.ds(off, chunk)], idx_v)
            pltpu.async_copy(table_s.at[idx_v], rows_v, sem).wait()
            pltpu.sync_copy(rows_v, out_hbm.at[pl.ds(off, chunk)])
            return carry

        lax.fori_loop(0, n_chunks, body, 0)

    return gather_kernel


def kernel(x, emb_table):
    b, s = x.shape
    v, d = emb_table.shape
    idx = x.astype(jnp.int32).reshape(-1)
    out = _make_gather(idx.shape[0], v, d)(idx, emb_table)
    return out.reshape(b, s, d)


# speedup vs baseline: 8.9347x; 1.7588x over previous
import functools

import jax
import jax.numpy as jnp
from jax import lax
from jax.experimental import pallas as pl
from jax.experimental.pallas import tpu as pltpu
from jax.experimental.pallas import tpu_sc as plsc

_K = 4
_CHUNK = 128


@functools.lru_cache(maxsize=None)
def _make_gather(n_total, n_var, d_model):
    info = plsc.get_sparse_core_info()
    nc, ns = info.num_cores, info.num_subcores
    nw = nc * ns

    chunk = _CHUNK
    per_w = n_total // nw
    n_chunks = per_w // chunk
    n_groups = n_chunks // _K
    assert per_w * nw == n_total and n_groups * _K * chunk == per_w

    mesh = plsc.VectorSubcoreMesh(core_axis_name="c", subcore_axis_name="s")

    @functools.partial(
        pl.kernel,
        mesh=mesh,
        out_type=jax.ShapeDtypeStruct((n_total, d_model), jnp.float32),
        scratch_types=[pltpu.VMEM((chunk,), jnp.int32) for _ in range(_K)]
        + [
            pltpu.VMEM((_K, chunk, d_model), jnp.float32),
            pltpu.VMEM_SHARED((n_var, d_model), jnp.float32),
        ]
        + [pltpu.SemaphoreType.DMA for _ in range(3 * _K)],
    )
    def gather_kernel(idx_hbm, table_hbm, out_hbm, *refs):
        idx_v = refs[0:_K]
        rows_v, table_s = refs[_K], refs[_K + 1]
        sems = refs[_K + 2 :]
        isem = sems[0:_K]
        gsem = sems[_K : 2 * _K]
        osem = sems[2 * _K : 3 * _K]

        sid = lax.axis_index("s")
        wid = sid * nc + lax.axis_index("c")
        base = wid * per_w

        @pl.when(sid == 0)
        def _stage():
            pltpu.sync_copy(table_hbm, table_s)

        plsc.subcore_barrier()

        def idx_src(j):
            return idx_hbm.at[pl.ds(base + j * chunk, chunk)]

        def out_dst(j):
            return out_hbm.at[pl.ds(base + j * chunk, chunk)]

        def group(g, carry):
            i0 = g * _K
            for b in range(_K):
                pltpu.async_copy(idx_src(i0 + b), idx_v[b], isem[b])
            for b in range(_K):
                pltpu.make_async_copy(idx_src(i0 + b), idx_v[b], isem[b]).wait()
                pltpu.async_copy(table_s.at[idx_v[b]], rows_v.at[b], gsem[b])
            for b in range(_K):
                pltpu.make_async_copy(
                    table_s.at[idx_v[b]], rows_v.at[b], gsem[b]
                ).wait()
                pltpu.async_copy(rows_v.at[b], out_dst(i0 + b), osem[b])
            for b in range(_K):
                pltpu.make_async_copy(
                    rows_v.at[b], out_dst(i0 + b), osem[b]
                ).wait()
            return carry

        lax.fori_loop(0, n_groups, group, 0)

    return gather_kernel


def kernel(x, emb_table):
    b, s = x.shape
    v, d = emb_table.shape
    idx = x.astype(jnp.int32).reshape(-1)
    out = _make_gather(idx.shape[0], v, d)(idx, emb_table)
    return out.reshape(b, s, d)


# speedup vs baseline: 9.2389x; 1.0340x over previous
import functools

import jax
import jax.numpy as jnp
from jax import lax
from jax.experimental import pallas as pl
from jax.experimental.pallas import tpu as pltpu
from jax.experimental.pallas import tpu_sc as plsc

_K = 5
_CHUNK = 128


@functools.lru_cache(maxsize=None)
def _make_gather(n_total, n_var, d_model):
    info = plsc.get_sparse_core_info()
    nc, ns = info.num_cores, info.num_subcores
    nw = nc * ns

    chunk = _CHUNK
    per_w = n_total // nw
    n_chunks = per_w // chunk
    n_groups = n_chunks // _K
    assert per_w * nw == n_total and n_groups * _K * chunk == per_w

    mesh = plsc.VectorSubcoreMesh(core_axis_name="c", subcore_axis_name="s")

    @functools.partial(
        pl.kernel,
        mesh=mesh,
        out_type=jax.ShapeDtypeStruct((n_total, d_model), jnp.float32),
        scratch_types=[pltpu.VMEM((chunk,), jnp.int32) for _ in range(_K)]
        + [
            pltpu.VMEM((_K, chunk, d_model), jnp.float32),
            pltpu.VMEM_SHARED((n_var, d_model), jnp.float32),
        ]
        + [pltpu.SemaphoreType.DMA for _ in range(3 * _K)],
    )
    def gather_kernel(idx_hbm, table_hbm, out_hbm, *refs):
        idx_v = refs[0:_K]
        rows_v, table_s = refs[_K], refs[_K + 1]
        sems = refs[_K + 2 :]
        isem = sems[0:_K]
        gsem = sems[_K : 2 * _K]
        osem = sems[2 * _K : 3 * _K]

        sid = lax.axis_index("s")
        wid = sid * nc + lax.axis_index("c")
        base = wid * per_w

        @pl.when(sid == 0)
        def _stage():
            pltpu.sync_copy(table_hbm, table_s)

        plsc.subcore_barrier()

        def idx_src(j):
            return idx_hbm.at[pl.ds(base + j * chunk, chunk)]

        def out_dst(j):
            return out_hbm.at[pl.ds(base + j * chunk, chunk)]

        def group(g, carry):
            i0 = g * _K
            for b in range(_K):
                pltpu.async_copy(idx_src(i0 + b), idx_v[b], isem[b])
            for b in range(_K):
                pltpu.make_async_copy(idx_src(i0 + b), idx_v[b], isem[b]).wait()
                pltpu.async_copy(table_s.at[idx_v[b]], rows_v.at[b], gsem[b])
            for b in range(_K):
                pltpu.make_async_copy(
                    table_s.at[idx_v[b]], rows_v.at[b], gsem[b]
                ).wait()
                pltpu.async_copy(rows_v.at[b], out_dst(i0 + b), osem[b])
            for b in range(_K):
                pltpu.make_async_copy(
                    rows_v.at[b], out_dst(i0 + b), osem[b]
                ).wait()
            return carry

        lax.fori_loop(0, n_groups, group, 0)

    return gather_kernel


def kernel(x, emb_table):
    b, s = x.shape
    v, d = emb_table.shape
    idx = x.astype(jnp.int32).reshape(-1)
    out = _make_gather(idx.shape[0], v, d)(idx, emb_table)
    return out.reshape(b, s, d)
